# baseline (speedup 1.0000x reference)
import functools

import jax
import jax.numpy as jnp
from jax import lax
from jax.experimental import pallas as pl
from jax.experimental.pallas import tpu as pltpu
from jax.experimental.pallas import tpu_sc as plsc

N = 10000
E = 320000
R = 8
D = 128

NC = 2
NS = 16
NW = NC * NS

EROWS = 2560
EPAD = EROWS * 128
RA = EROWS // NS
RB = EROWS // NW
CT = 81920

_mesh = plsc.VectorSubcoreMesh(core_axis_name="c", subcore_axis_name="s")


@functools.partial(
    pl.kernel,
    out_type=(
        jax.ShapeDtypeStruct((EROWS, 128), jnp.int32),
        jax.ShapeDtypeStruct((EROWS, 128), jnp.float32),
    ),
    mesh=_mesh,
    scratch_types=[
        pltpu.VMEM_SHARED((CT, 16), jnp.float32),
        pltpu.VMEM((16, 128), jnp.int32),
        pltpu.VMEM((16, 128), jnp.int32),
        pltpu.VMEM((16, 128), jnp.int32),
        pltpu.VMEM((16, 128), jnp.int32),
        pltpu.VMEM((16, 128), jnp.int32),
        pltpu.VMEM((16, 128), jnp.float32),
        pltpu.VMEM((128, 16), jnp.float32),
        pltpu.VMEM((128, 16), jnp.float32),
        pltpu.VMEM((128, 16), jnp.float32),
    ],
    compiler_params=pltpu.CompilerParams(use_tc_tiling_on_sc=False),
)
def _phase_a(dst_hbm, rel_hbm, src_hbm, g_hbm, w_hbm,
             cnt_sh, dstb, relb, srcb, keyb, gb, wb, ones, crow, zbuf):
    c = lax.axis_index("c")
    s = lax.axis_index("s")

    one16 = jnp.ones((16,), jnp.float32)
    zero16 = jnp.zeros((16,), jnp.float32)

    def _fill(i, _):
        ones[i] = one16
        zbuf[i] = zero16
        return 0
    lax.fori_loop(0, 128, _fill, 0)

    def _zc(i, _):
        pltpu.sync_copy(zbuf, cnt_sh.at[pl.ds(s * 5120 + i * 128, 128)])
        return 0
    lax.fori_loop(0, 40, _zc, 0)
    plsc.subcore_barrier()

    def _batch1(bi, _):
        row0 = s * RA + bi * 16
        pltpu.sync_copy(dst_hbm.at[pl.ds(row0, 16)], dstb)
        pltpu.sync_copy(rel_hbm.at[pl.ds(row0, 16)], relb)
        pltpu.sync_copy(src_hbm.at[pl.ds(row0, 16)], srcb)

        def _keys(i, _):
            j = i // 8
            col = (i % 8) * 16
            d = dstb[j, pl.ds(col, 16)]
            r = relb[j, pl.ds(col, 16)]
            sr = srcb[j, pl.ds(col, 16)]
            keyb[j, pl.ds(col, 16)] = d * R + r
            gb[j, pl.ds(col, 16)] = r * N + sr
            return 0
        lax.fori_loop(0, 128, _keys, 0)

        @pl.when(c == 0)
        def _():
            pltpu.sync_copy(gb, g_hbm.at[pl.ds(row0, 16)])

        def _scat(j, _):
            pltpu.sync_copy(ones, cnt_sh.at[keyb.at[j]], add=True)
            return 0
        lax.fori_loop(0, 16, _scat, 0)
        return 0
    lax.fori_loop(0, RA // 16, _batch1, 0)
    plsc.subcore_barrier()

    wid = s * NC + c
    lane = lax.iota(jnp.int32, 16)
    onehot = [jnp.where(lane == k, 1.0, 0.0).astype(jnp.float32)
              for k in range(16)]

    def _batch2(bi, _):
        row0 = wid * RB + bi * 16
        pltpu.sync_copy(dst_hbm.at[pl.ds(row0, 16)], dstb)
        pltpu.sync_copy(rel_hbm.at[pl.ds(row0, 16)], relb)

        def _keys2(i, _):
            j = i // 8
            col = (i % 8) * 16
            keyb[j, pl.ds(col, 16)] = (dstb[j, pl.ds(col, 16)] * R
                                       + relb[j, pl.ds(col, 16)])
            return 0
        lax.fori_loop(0, 128, _keys2, 0)

        def _row(j, _):
            pltpu.sync_copy(cnt_sh.at[keyb.at[j]], crow)

            def _w(grp, _):
                cv = jnp.zeros((16,), jnp.float32)
                for k in range(16):
                    cv = cv + crow[grp * 16 + k] * onehot[k]
                d = dstb[j, pl.ds(grp * 16, 16)]
                wb[j, pl.ds(grp * 16, 16)] = jnp.where(
                    d < N, 1.0 / jnp.maximum(cv, 1.0), 0.0)
                return 0
            lax.fori_loop(0, 8, _w, 0)
            return 0
        lax.fori_loop(0, 16, _row, 0)

        pltpu.sync_copy(wb, w_hbm.at[pl.ds(row0, 16)])
        return 0
    lax.fori_loop(0, RB // 16, _batch2, 0)


@functools.partial(
    pl.kernel,
    out_type=jax.ShapeDtypeStruct((NC * N, 128), jnp.float32),
    mesh=_mesh,
    scratch_types=[
        pltpu.VMEM_SHARED((N, 128), jnp.float32),
        pltpu.VMEM((32, 32), jnp.int32),
        pltpu.VMEM((32, 32), jnp.int32),
        pltpu.VMEM((32, 32), jnp.float32),
        pltpu.VMEM((32, 64), jnp.int32),
        pltpu.VMEM((32, 64), jnp.int32),
        pltpu.VMEM((32, 64), jnp.int32),
        pltpu.VMEM((32, 64), jnp.int32),
        pltpu.VMEM((32, 128), jnp.float32),
        pltpu.VMEM((32, 128), jnp.float32),
        pltpu.VMEM((32, 128), jnp.float32),
        pltpu.VMEM((32, 128), jnp.float32),
        pltpu.VMEM((25, 128), jnp.float32),
        pltpu.SemaphoreType.DMA,
        pltpu.SemaphoreType.DMA,
        pltpu.SemaphoreType.DMA,
        pltpu.SemaphoreType.DMA,
        pltpu.SemaphoreType.DMA,
        pltpu.SemaphoreType.DMA,
        pltpu.SemaphoreType.DMA,
        pltpu.SemaphoreType.DMA,
    ],
    compiler_params=pltpu.CompilerParams(use_tc_tiling_on_sc=False,
                                         needs_layout_passes=False),
)
def _phase_b(h_hbm, g_hbm, dst_hbm, w_hbm, out_hbm,
             acc, gb, db, wbuf, bf0, bf1, bf2, bf3, fb0, fb1, fb2, fb3, zbuf,
             semg0, semg1, semg2, semg3, semsc0, semsc1, semsc2, semsc3):
    c = lax.axis_index("c")
    s = lax.axis_index("s")
    wid = s * NC + c

    zero16 = jnp.zeros((16,), jnp.float32)

    def _zf(i, _):
        j = i // 8
        col = (i % 8) * 16
        zbuf[j, pl.ds(col, 16)] = zero16
        return 0
    lax.fori_loop(0, 200, _zf, 0)

    def _zc(i, _):
        pltpu.sync_copy(zbuf, acc.at[pl.ds(s * 625 + i * 25, 25)])
        return 0
    lax.fori_loop(0, 25, _zc, 0)

    plsc.subcore_barrier()

    bfs = (bf0, bf1, bf2, bf3)
    fbs = (fb0, fb1, fb2, fb3)
    semgs = (semg0, semg1, semg2, semg3)
    semscs = (semsc0, semsc1, semsc2, semsc3)
    himask = jnp.full((16,), -65536, jnp.int32)

    def _scale(q, c):
        bf = bfs[q]
        fb = fbs[q]

        def _grp(grp, _):
            wv = wbuf[c, pl.ds(grp * 16, 16)]
            for e16 in range(16):
                e = grp * 16 + e16
                ws = wv[e16]
                for k in range(4):
                    w32 = bf[e, pl.ds(16 * k, 16)]
                    lo = plsc.bitcast(jnp.left_shift(w32, 16), jnp.float32)
                    hi = plsc.bitcast(jnp.bitwise_and(w32, himask),
                                      jnp.float32)
                    fb[e, pl.ds(32 * k, 16)] = lo * ws
                    fb[e, pl.ds(32 * k + 16, 16)] = hi * ws
            return 0
        lax.fori_loop(0, 2, _grp, 0)

    def _gather(c, q):
        pltpu.async_copy(h_hbm.at[gb.at[c]], bfs[q], semgs[q])

    def _wait_gather(c, q):
        pltpu.make_async_copy(h_hbm.at[gb.at[c]], bfs[q], semgs[q]).wait()

    def _scatter(c, q):
        pltpu.async_copy(fbs[q], acc.at[db.at[c]], semscs[q], add=True)

    def _wait_scat(q):
        pltpu.make_async_copy(fbs[q], acc.at[db.at[0]], semscs[q]).wait()

    def _batch(bi, _):
        r0b = wid * 4 * RB + bi * 32
        pltpu.sync_copy(g_hbm.at[pl.ds(r0b, 32)], gb)
        pltpu.sync_copy(dst_hbm.at[pl.ds(r0b, 32)], db)
        pltpu.sync_copy(w_hbm.at[pl.ds(r0b, 32)], wbuf)

        _gather(0, 0)
        _gather(1, 1)

        def _quad(qi, _):
            c0 = 4 * qi
            for q in range(4):
                c = c0 + q
                _wait_gather(c, q)

                @pl.when(qi >= 1)
                def _():
                    _wait_scat(q)
                _scale(q, c)
                _scatter(c, q)
                qn = (q + 2) % 4
                if q < 2:
                    _gather(c + 2, qn)
                else:
                    @pl.when(qi < 7)
                    def _():
                        _gather(c + 2, qn)
            return 0
        lax.fori_loop(0, 8, _quad, 0)
        _wait_scat(0)
        _wait_scat(1)
        _wait_scat(2)
        _wait_scat(3)
        return 0
    lax.fori_loop(0, RB // 8, _batch, 0)
    plsc.subcore_barrier()

    def _wb(i, _):
        pltpu.sync_copy(acc.at[pl.ds(s * 625 + i * 125, 125)],
                        out_hbm.at[pl.ds(c * N + s * 625 + i * 125, 125)])
        return 0
    lax.fori_loop(0, 5, _wb, 0)


_BL = 2000
_NBL = N // _BL


def _h_body(x_ref, w_ref, o_ref):
    o_ref[...] = jnp.dot(x_ref[...], w_ref[0],
                         preferred_element_type=jnp.float32
                         ).astype(jnp.bfloat16)


_build_h = pl.pallas_call(
    _h_body,
    grid=(R, _NBL),
    in_specs=[
        pl.BlockSpec((_BL, D), lambda r, i: (i, 0)),
        pl.BlockSpec((1, D, D), lambda r, i: (r, 0, 0)),
    ],
    out_specs=pl.BlockSpec((_BL, D), lambda r, i: (r * _NBL + i, 0)),
    out_shape=jax.ShapeDtypeStruct((R * N, D), jnp.bfloat16),
)

_COLPERM = [32 * g + off + t
            for g in range(4) for t in range(16) for off in (0, 16)]


def _make_combine(relu):
    def _body(acc_ref0, acc_ref1, x_ref, root_ref, b_ref, o_ref):
        v = (acc_ref0[...] + acc_ref1[...]
             + jnp.dot(x_ref[...], root_ref[...],
                       preferred_element_type=jnp.float32)
             + b_ref[...])
        o_ref[...] = jnp.maximum(v, 0.0) if relu else v

    return pl.pallas_call(
        _body,
        grid=(_NBL,),
        in_specs=[
            pl.BlockSpec((_BL, D), lambda i: (i, 0)),
            pl.BlockSpec((_BL, D), lambda i: (_NBL + i, 0)),
            pl.BlockSpec((_BL, D), lambda i: (i, 0)),
            pl.BlockSpec((D, D), lambda i: (0, 0)),
            pl.BlockSpec((1, D), lambda i: (0, 0)),
        ],
        out_specs=pl.BlockSpec((_BL, D), lambda i: (i, 0)),
        out_shape=jax.ShapeDtypeStruct((N, D), jnp.float32),
    )


_combine_relu = _make_combine(True)
_combine_lin = _make_combine(False)


@jax.jit
def kernel(x, edge_index, edge_type, W0, root0, b0, W1, root1, b1):
    src = edge_index[0]
    dst = edge_index[1]
    npad = EPAD - E
    dst_a = jnp.concatenate(
        [dst, jnp.full((npad,), N, jnp.int32)]).reshape(EROWS, 128)
    dst_b = jnp.concatenate(
        [dst, jnp.zeros((npad,), jnp.int32)]).reshape(EROWS, 128)
    rel_p = jnp.concatenate(
        [edge_type, jnp.zeros((npad,), jnp.int32)]).reshape(EROWS, 128)
    src_p = jnp.concatenate(
        [src, jnp.zeros((npad,), jnp.int32)]).reshape(EROWS, 128)
    b0r = b0.reshape(1, D)
    b1r = b1.reshape(1, D)

    g2d, w2d = _phase_a(dst_a, rel_p, src_p)
    g32 = g2d.reshape(EROWS * 4, 32)
    w32 = w2d.reshape(EROWS * 4, 32)
    dst32 = dst_b.reshape(EROWS * 4, 32)
    cperm = jnp.asarray(_COLPERM, jnp.int32)
    W0p = W0[:, :, cperm]
    W1p = W1[:, :, cperm]

    def _as_i32(h):
        return jax.lax.bitcast_convert_type(
            h.reshape(R * N, 64, 2), jnp.int32)

    h0 = _build_h(x, W0p)
    acc0 = _phase_b(_as_i32(h0), g32, dst32, w32)
    x1 = _combine_relu(acc0, acc0, x, root0, b0r)

    h1 = _build_h(x1, W1p)
    acc1 = _phase_b(_as_i32(h1), g32, dst32, w32)
    out = _combine_lin(acc1, acc1, x1, root1, b1r)
    return out


# speedup vs baseline: 1.0245x; 1.0245x over previous
import functools

import jax
import jax.numpy as jnp
from jax import lax
from jax.experimental import pallas as pl
from jax.experimental.pallas import tpu as pltpu
from jax.experimental.pallas import tpu_sc as plsc

N = 10000
E = 320000
R = 8
D = 128

NC = 2
NS = 16
NW = NC * NS

EROWS = 2560
EPAD = EROWS * 128
RA = EROWS // NS
RB = EROWS // NW
CT = 81920

_mesh = plsc.VectorSubcoreMesh(core_axis_name="c", subcore_axis_name="s")


@functools.partial(
    pl.kernel,
    out_type=(
        jax.ShapeDtypeStruct((EROWS, 128), jnp.int32),
        jax.ShapeDtypeStruct((EROWS, 128), jnp.float32),
    ),
    mesh=_mesh,
    scratch_types=[
        pltpu.VMEM_SHARED((CT, 16), jnp.float32),
        pltpu.VMEM((16, 128), jnp.int32),
        pltpu.VMEM((16, 128), jnp.int32),
        pltpu.VMEM((16, 128), jnp.int32),
        pltpu.VMEM((16, 128), jnp.int32),
        pltpu.VMEM((16, 128), jnp.int32),
        pltpu.VMEM((16, 128), jnp.float32),
        pltpu.VMEM((128, 16), jnp.float32),
        pltpu.VMEM((128, 16), jnp.float32),
        pltpu.VMEM((128, 16), jnp.float32),
    ],
    compiler_params=pltpu.CompilerParams(use_tc_tiling_on_sc=False),
)
def _phase_a(dst_hbm, rel_hbm, src_hbm, g_hbm, w_hbm,
             cnt_sh, dstb, relb, srcb, keyb, gb, wb, ones, crow, zbuf):
    c = lax.axis_index("c")
    s = lax.axis_index("s")

    one16 = jnp.ones((16,), jnp.float32)
    zero16 = jnp.zeros((16,), jnp.float32)

    def _fill(i, _):
        ones[i] = one16
        zbuf[i] = zero16
        return 0
    lax.fori_loop(0, 128, _fill, 0)

    def _zc(i, _):
        pltpu.sync_copy(zbuf, cnt_sh.at[pl.ds(s * 5120 + i * 128, 128)])
        return 0
    lax.fori_loop(0, 40, _zc, 0)
    plsc.subcore_barrier()

    def _batch1(bi, _):
        row0 = s * RA + bi * 16
        pltpu.sync_copy(dst_hbm.at[pl.ds(row0, 16)], dstb)
        pltpu.sync_copy(rel_hbm.at[pl.ds(row0, 16)], relb)
        pltpu.sync_copy(src_hbm.at[pl.ds(row0, 16)], srcb)

        def _keys(i, _):
            j = i // 8
            col = (i % 8) * 16
            d = dstb[j, pl.ds(col, 16)]
            r = relb[j, pl.ds(col, 16)]
            sr = srcb[j, pl.ds(col, 16)]
            keyb[j, pl.ds(col, 16)] = d * R + r
            gb[j, pl.ds(col, 16)] = r * N + sr
            return 0
        lax.fori_loop(0, 128, _keys, 0)

        @pl.when(c == 0)
        def _():
            pltpu.sync_copy(gb, g_hbm.at[pl.ds(row0, 16)])

        def _scat(j, _):
            pltpu.sync_copy(ones, cnt_sh.at[keyb.at[j]], add=True)
            return 0
        lax.fori_loop(0, 16, _scat, 0)
        return 0
    lax.fori_loop(0, RA // 16, _batch1, 0)
    plsc.subcore_barrier()

    wid = s * NC + c
    lane = lax.iota(jnp.int32, 16)
    onehot = [jnp.where(lane == k, 1.0, 0.0).astype(jnp.float32)
              for k in range(16)]

    def _batch2(bi, _):
        row0 = wid * RB + bi * 16
        pltpu.sync_copy(dst_hbm.at[pl.ds(row0, 16)], dstb)
        pltpu.sync_copy(rel_hbm.at[pl.ds(row0, 16)], relb)

        def _keys2(i, _):
            j = i // 8
            col = (i % 8) * 16
            keyb[j, pl.ds(col, 16)] = (dstb[j, pl.ds(col, 16)] * R
                                       + relb[j, pl.ds(col, 16)])
            return 0
        lax.fori_loop(0, 128, _keys2, 0)

        def _row(j, _):
            pltpu.sync_copy(cnt_sh.at[keyb.at[j]], crow)

            def _w(grp, _):
                cv = jnp.zeros((16,), jnp.float32)
                for k in range(16):
                    cv = cv + crow[grp * 16 + k] * onehot[k]
                d = dstb[j, pl.ds(grp * 16, 16)]
                wb[j, pl.ds(grp * 16, 16)] = jnp.where(
                    d < N, 1.0 / jnp.maximum(cv, 1.0), 0.0)
                return 0
            lax.fori_loop(0, 8, _w, 0)
            return 0
        lax.fori_loop(0, 16, _row, 0)

        pltpu.sync_copy(wb, w_hbm.at[pl.ds(row0, 16)])
        return 0
    lax.fori_loop(0, RB // 16, _batch2, 0)


@functools.partial(
    pl.kernel,
    out_type=jax.ShapeDtypeStruct((NC * N, 128), jnp.float32),
    mesh=_mesh,
    scratch_types=[
        pltpu.VMEM_SHARED((N, 128), jnp.float32),
        pltpu.VMEM((16, 64), jnp.int32),
        pltpu.VMEM((16, 64), jnp.int32),
        pltpu.VMEM((16, 64), jnp.float32),
        pltpu.VMEM((64, 64), jnp.int32),
        pltpu.VMEM((64, 64), jnp.int32),
        pltpu.VMEM((64, 128), jnp.float32),
        pltpu.VMEM((64, 128), jnp.float32),
        pltpu.VMEM((25, 128), jnp.float32),
        pltpu.SemaphoreType.DMA,
        pltpu.SemaphoreType.DMA,
        pltpu.SemaphoreType.DMA,
        pltpu.SemaphoreType.DMA,
    ],
    compiler_params=pltpu.CompilerParams(use_tc_tiling_on_sc=False,
                                         needs_layout_passes=False),
)
def _phase_b(h_hbm, g_hbm, dst_hbm, w_hbm, out_hbm,
             acc, gb, db, wbuf, bf0, bf1, fb0, fb1, zbuf,
             semg0, semg1, semsc0, semsc1):
    c = lax.axis_index("c")
    s = lax.axis_index("s")
    wid = s * NC + c

    zero16 = jnp.zeros((16,), jnp.float32)

    def _zf(i, _):
        j = i // 8
        col = (i % 8) * 16
        zbuf[j, pl.ds(col, 16)] = zero16
        return 0
    lax.fori_loop(0, 200, _zf, 0)

    def _zc(i, _):
        pltpu.sync_copy(zbuf, acc.at[pl.ds(s * 625 + i * 25, 25)])
        return 0
    lax.fori_loop(0, 25, _zc, 0)

    plsc.subcore_barrier()

    bfs = (bf0, bf1)
    fbs = (fb0, fb1)
    semgs = (semg0, semg1)
    semscs = (semsc0, semsc1)
    himask = jnp.full((16,), -65536, jnp.int32)

    def _scale(q, c):
        bf = bfs[q]
        fb = fbs[q]

        def _grp(grp, _):
            wv = wbuf[c, pl.ds(grp * 16, 16)]
            for e16 in range(16):
                e = grp * 16 + e16
                ws = wv[e16]
                for k in range(4):
                    w32 = bf[e, pl.ds(16 * k, 16)]
                    lo = plsc.bitcast(jnp.left_shift(w32, 16), jnp.float32)
                    hi = plsc.bitcast(jnp.bitwise_and(w32, himask),
                                      jnp.float32)
                    fb[e, pl.ds(32 * k, 16)] = lo * ws
                    fb[e, pl.ds(32 * k + 16, 16)] = hi * ws
            return 0
        lax.fori_loop(0, 4, _grp, 0)

    def _gather(c, q):
        pltpu.async_copy(h_hbm.at[gb.at[c]], bfs[q], semgs[q])

    def _wait_gather(c, q):
        pltpu.make_async_copy(h_hbm.at[gb.at[c]], bfs[q], semgs[q]).wait()

    def _scatter(c, q):
        pltpu.async_copy(fbs[q], acc.at[db.at[c]], semscs[q], add=True)

    def _wait_scat(q):
        pltpu.make_async_copy(fbs[q], acc.at[db.at[0]], semscs[q]).wait()

    def _batch(bi, _):
        r0b = 2 * wid * RB + bi * 16
        pltpu.sync_copy(g_hbm.at[pl.ds(r0b, 16)], gb)
        pltpu.sync_copy(dst_hbm.at[pl.ds(r0b, 16)], db)
        pltpu.sync_copy(w_hbm.at[pl.ds(r0b, 16)], wbuf)

        _gather(0, 0)

        def _pair(pi, _):
            c0 = 2 * pi
            _wait_gather(c0, 0)
            _gather(c0 + 1, 1)

            @pl.when(pi >= 1)
            def _():
                _wait_scat(0)
            _scale(0, c0)
            _scatter(c0, 0)

            _wait_gather(c0 + 1, 1)

            @pl.when(pi < 7)
            def _():
                _gather(c0 + 2, 0)

            @pl.when(pi >= 1)
            def _():
                _wait_scat(1)
            _scale(1, c0 + 1)
            _scatter(c0 + 1, 1)
            return 0
        lax.fori_loop(0, 8, _pair, 0)
        _wait_scat(0)
        _wait_scat(1)
        return 0
    lax.fori_loop(0, RB // 8, _batch, 0)
    plsc.subcore_barrier()

    def _wb(i, _):
        pltpu.sync_copy(acc.at[pl.ds(s * 625 + i * 125, 125)],
                        out_hbm.at[pl.ds(c * N + s * 625 + i * 125, 125)])
        return 0
    lax.fori_loop(0, 5, _wb, 0)


_BL = 2000
_NBL = N // _BL


def _h_body(x_ref, w_ref, o_ref):
    o_ref[...] = jnp.dot(x_ref[...], w_ref[0],
                         preferred_element_type=jnp.float32
                         ).astype(jnp.bfloat16)


_build_h = pl.pallas_call(
    _h_body,
    grid=(R, _NBL),
    in_specs=[
        pl.BlockSpec((_BL, D), lambda r, i: (i, 0)),
        pl.BlockSpec((1, D, D), lambda r, i: (r, 0, 0)),
    ],
    out_specs=pl.BlockSpec((_BL, D), lambda r, i: (r * _NBL + i, 0)),
    out_shape=jax.ShapeDtypeStruct((R * N, D), jnp.bfloat16),
)

_COLPERM = [32 * g + off + t
            for g in range(4) for t in range(16) for off in (0, 16)]


def _make_combine(relu):
    def _body(acc_ref0, acc_ref1, x_ref, root_ref, b_ref, o_ref):
        v = (acc_ref0[...] + acc_ref1[...]
             + jnp.dot(x_ref[...], root_ref[...],
                       preferred_element_type=jnp.float32)
             + b_ref[...])
        o_ref[...] = jnp.maximum(v, 0.0) if relu else v

    return pl.pallas_call(
        _body,
        grid=(_NBL,),
        in_specs=[
            pl.BlockSpec((_BL, D), lambda i: (i, 0)),
            pl.BlockSpec((_BL, D), lambda i: (_NBL + i, 0)),
            pl.BlockSpec((_BL, D), lambda i: (i, 0)),
            pl.BlockSpec((D, D), lambda i: (0, 0)),
            pl.BlockSpec((1, D), lambda i: (0, 0)),
        ],
        out_specs=pl.BlockSpec((_BL, D), lambda i: (i, 0)),
        out_shape=jax.ShapeDtypeStruct((N, D), jnp.float32),
    )


_combine_relu = _make_combine(True)
_combine_lin = _make_combine(False)


@jax.jit
def kernel(x, edge_index, edge_type, W0, root0, b0, W1, root1, b1):
    src = edge_index[0]
    dst = edge_index[1]
    npad = EPAD - E
    dst_a = jnp.concatenate(
        [dst, jnp.full((npad,), N, jnp.int32)]).reshape(EROWS, 128)
    dst_b = jnp.concatenate(
        [dst, jnp.zeros((npad,), jnp.int32)]).reshape(EROWS, 128)
    rel_p = jnp.concatenate(
        [edge_type, jnp.zeros((npad,), jnp.int32)]).reshape(EROWS, 128)
    src_p = jnp.concatenate(
        [src, jnp.zeros((npad,), jnp.int32)]).reshape(EROWS, 128)
    b0r = b0.reshape(1, D)
    b1r = b1.reshape(1, D)

    g2d, w2d = _phase_a(dst_a, rel_p, src_p)
    g32 = g2d.reshape(EROWS * 2, 64)
    w32 = w2d.reshape(EROWS * 2, 64)
    dst32 = dst_b.reshape(EROWS * 2, 64)
    cperm = jnp.asarray(_COLPERM, jnp.int32)
    W0p = W0[:, :, cperm]
    W1p = W1[:, :, cperm]

    def _as_i32(h):
        return jax.lax.bitcast_convert_type(
            h.reshape(R * N, 64, 2), jnp.int32)

    h0 = _build_h(x, W0p)
    acc0 = _phase_b(_as_i32(h0), g32, dst32, w32)
    x1 = _combine_relu(acc0, acc0, x, root0, b0r)

    h1 = _build_h(x1, W1p)
    acc1 = _phase_b(_as_i32(h1), g32, dst32, w32)
    out = _combine_lin(acc1, acc1, x1, root1, b1r)
    return out


# speedup vs baseline: 1.2296x; 1.2002x over previous
import functools

import jax
import jax.numpy as jnp
from jax import lax
from jax.experimental import pallas as pl
from jax.experimental.pallas import tpu as pltpu
from jax.experimental.pallas import tpu_sc as plsc

N = 10000
E = 320000
R = 8
D = 128

NC = 2
NS = 16
NW = NC * NS

EROWS = 2560
EPAD = EROWS * 128
RA = EROWS // NS
RB = EROWS // NW
CT = 81920

_mesh = plsc.VectorSubcoreMesh(core_axis_name="c", subcore_axis_name="s")


@functools.partial(
    pl.kernel,
    out_type=(
        jax.ShapeDtypeStruct((EROWS, 128), jnp.int32),
        jax.ShapeDtypeStruct((EROWS, 128), jnp.float32),
    ),
    mesh=_mesh,
    scratch_types=[
        pltpu.VMEM_SHARED((CT, 16), jnp.float32),
        pltpu.VMEM((16, 128), jnp.int32),
        pltpu.VMEM((16, 128), jnp.int32),
        pltpu.VMEM((16, 128), jnp.int32),
        pltpu.VMEM((16, 128), jnp.int32),
        pltpu.VMEM((16, 128), jnp.int32),
        pltpu.VMEM((16, 128), jnp.float32),
        pltpu.VMEM((128, 16), jnp.float32),
        pltpu.VMEM((128, 16), jnp.float32),
        pltpu.VMEM((128, 16), jnp.float32),
    ],
    compiler_params=pltpu.CompilerParams(use_tc_tiling_on_sc=False),
)
def _phase_a(dst_hbm, rel_hbm, src_hbm, g_hbm, w_hbm,
             cnt_sh, dstb, relb, srcb, keyb, gb, wb, ones, crow, zbuf):
    c = lax.axis_index("c")
    s = lax.axis_index("s")

    one16 = jnp.ones((16,), jnp.float32)
    zero16 = jnp.zeros((16,), jnp.float32)

    def _fill(i, _):
        ones[i] = one16
        zbuf[i] = zero16
        return 0
    lax.fori_loop(0, 128, _fill, 0)

    def _zc(i, _):
        pltpu.sync_copy(zbuf, cnt_sh.at[pl.ds(s * 5120 + i * 128, 128)])
        return 0
    lax.fori_loop(0, 40, _zc, 0)
    plsc.subcore_barrier()

    def _batch1(bi, _):
        row0 = s * RA + bi * 16
        pltpu.sync_copy(dst_hbm.at[pl.ds(row0, 16)], dstb)
        pltpu.sync_copy(rel_hbm.at[pl.ds(row0, 16)], relb)
        pltpu.sync_copy(src_hbm.at[pl.ds(row0, 16)], srcb)

        def _keys(i, _):
            j = i // 8
            col = (i % 8) * 16
            d = dstb[j, pl.ds(col, 16)]
            r = relb[j, pl.ds(col, 16)]
            sr = srcb[j, pl.ds(col, 16)]
            keyb[j, pl.ds(col, 16)] = d * R + r
            gb[j, pl.ds(col, 16)] = r * N + sr
            return 0
        lax.fori_loop(0, 128, _keys, 0)

        @pl.when(c == 0)
        def _():
            pltpu.sync_copy(gb, g_hbm.at[pl.ds(row0, 16)])

        def _scat(j, _):
            pltpu.sync_copy(ones, cnt_sh.at[keyb.at[j]], add=True)
            return 0
        lax.fori_loop(0, 16, _scat, 0)
        return 0
    lax.fori_loop(0, RA // 16, _batch1, 0)
    plsc.subcore_barrier()

    wid = s * NC + c
    lane = lax.iota(jnp.int32, 16)
    onehot = [jnp.where(lane == k, 1.0, 0.0).astype(jnp.float32)
              for k in range(16)]

    def _batch2(bi, _):
        row0 = wid * RB + bi * 16
        pltpu.sync_copy(dst_hbm.at[pl.ds(row0, 16)], dstb)
        pltpu.sync_copy(rel_hbm.at[pl.ds(row0, 16)], relb)

        def _keys2(i, _):
            j = i // 8
            col = (i % 8) * 16
            keyb[j, pl.ds(col, 16)] = (dstb[j, pl.ds(col, 16)] * R
                                       + relb[j, pl.ds(col, 16)])
            return 0
        lax.fori_loop(0, 128, _keys2, 0)

        def _row(j, _):
            pltpu.sync_copy(cnt_sh.at[keyb.at[j]], crow)

            def _w(grp, _):
                cv = jnp.zeros((16,), jnp.float32)
                for k in range(16):
                    cv = cv + crow[grp * 16 + k] * onehot[k]
                d = dstb[j, pl.ds(grp * 16, 16)]
                wb[j, pl.ds(grp * 16, 16)] = jnp.where(
                    d < N, 1.0 / jnp.maximum(cv, 1.0), 0.0)
                return 0
            lax.fori_loop(0, 8, _w, 0)
            return 0
        lax.fori_loop(0, 16, _row, 0)

        pltpu.sync_copy(wb, w_hbm.at[pl.ds(row0, 16)])
        return 0
    lax.fori_loop(0, RB // 16, _batch2, 0)


@functools.partial(
    pl.kernel,
    out_type=jax.ShapeDtypeStruct((NC * N, 128), jnp.float32),
    mesh=_mesh,
    scratch_types=[
        pltpu.VMEM_SHARED((N, 128), jnp.float32),
        pltpu.VMEM((32, 64), jnp.int32),
        pltpu.VMEM((32, 64), jnp.int32),
        pltpu.VMEM((32, 64), jnp.float32),
        pltpu.VMEM((64, 128), jnp.float32),
        pltpu.VMEM((64, 128), jnp.float32),
        pltpu.VMEM((64, 128), jnp.float32),
        pltpu.VMEM((64, 128), jnp.float32),
        pltpu.VMEM((25, 128), jnp.float32),
        pltpu.SemaphoreType.DMA,
        pltpu.SemaphoreType.DMA,
        pltpu.SemaphoreType.DMA,
        pltpu.SemaphoreType.DMA,
        pltpu.SemaphoreType.DMA,
        pltpu.SemaphoreType.DMA,
        pltpu.SemaphoreType.DMA,
        pltpu.SemaphoreType.DMA,
    ],
    compiler_params=pltpu.CompilerParams(use_tc_tiling_on_sc=False),
)
def _phase_b(h_hbm, g_hbm, dst_hbm, w_hbm, out_hbm,
             acc, gb, db, wbuf, rows0, rows1, rows2, rows3, zbuf,
             semg0, semg1, semg2, semg3, semsc0, semsc1, semsc2, semsc3):
    c = lax.axis_index("c")
    s = lax.axis_index("s")
    wid = s * NC + c

    zero16 = jnp.zeros((16,), jnp.float32)

    def _zf(i, _):
        j = i // 8
        col = (i % 8) * 16
        zbuf[j, pl.ds(col, 16)] = zero16
        return 0
    lax.fori_loop(0, 200, _zf, 0)

    def _zc(i, _):
        pltpu.sync_copy(zbuf, acc.at[pl.ds(s * 625 + i * 25, 25)])
        return 0
    lax.fori_loop(0, 25, _zc, 0)

    plsc.subcore_barrier()

    bufs = (rows0, rows1, rows2, rows3)
    semgs = (semg0, semg1, semg2, semg3)
    semscs = (semsc0, semsc1, semsc2, semsc3)

    def _scale(q, c):
        rows = bufs[q]

        def _grp(grp, _):
            wv = wbuf[c, pl.ds(grp * 16, 16)]
            for e16 in range(16):
                e = grp * 16 + e16
                ws = wv[e16]
                for cix in range(8):
                    rows[e, pl.ds(cix * 16, 16)] = (
                        rows[e, pl.ds(cix * 16, 16)] * ws)
            return 0
        lax.fori_loop(0, 4, _grp, 0)

    def _gather(c, q):
        pltpu.async_copy(h_hbm.at[gb.at[c]], bufs[q], semgs[q])

    def _wait_gather(c, q):
        pltpu.make_async_copy(h_hbm.at[gb.at[c]], bufs[q], semgs[q]).wait()

    def _scatter(c, q):
        pltpu.async_copy(bufs[q], acc.at[db.at[c]], semscs[q], add=True)

    def _wait_scat(q):
        pltpu.make_async_copy(bufs[q], acc.at[db.at[0]], semscs[q]).wait()

    def _batch(bi, _):
        r0b = 2 * (wid * RB + bi * 16)
        pltpu.sync_copy(g_hbm.at[pl.ds(r0b, 32)], gb)
        pltpu.sync_copy(dst_hbm.at[pl.ds(r0b, 32)], db)
        pltpu.sync_copy(w_hbm.at[pl.ds(r0b, 32)], wbuf)

        _gather(0, 0)
        _gather(1, 1)

        def _quad(qi, _):
            c0 = 4 * qi
            for q in range(4):
                c = c0 + q
                _wait_gather(c, q)
                _scale(q, c)
                _scatter(c, q)
                qn = (q + 2) % 4
                if q < 2:
                    @pl.when(qi >= 1)
                    def _():
                        _wait_scat(qn)
                    _gather(c + 2, qn)
                else:
                    @pl.when(qi < 7)
                    def _():
                        _wait_scat(qn)
                        _gather(c + 2, qn)
            return 0
        lax.fori_loop(0, 8, _quad, 0)
        _wait_scat(0)
        _wait_scat(1)
        _wait_scat(2)
        _wait_scat(3)
        return 0
    lax.fori_loop(0, RB // 16, _batch, 0)
    plsc.subcore_barrier()

    def _wb(i, _):
        pltpu.sync_copy(acc.at[pl.ds(s * 625 + i * 125, 125)],
                        out_hbm.at[pl.ds(c * N + s * 625 + i * 125, 125)])
        return 0
    lax.fori_loop(0, 5, _wb, 0)


_BL = 2000
_NBL = N // _BL


def _h_body(x_ref, w_ref, o_ref):
    o_ref[...] = jnp.dot(x_ref[...], w_ref[0],
                         preferred_element_type=jnp.float32)


_build_h = pl.pallas_call(
    _h_body,
    grid=(R, _NBL),
    in_specs=[
        pl.BlockSpec((_BL, D), lambda r, i: (i, 0)),
        pl.BlockSpec((1, D, D), lambda r, i: (r, 0, 0)),
    ],
    out_specs=pl.BlockSpec((_BL, D), lambda r, i: (r * _NBL + i, 0)),
    out_shape=jax.ShapeDtypeStruct((R * N, D), jnp.float32),
)


def _make_combine(relu):
    def _body(acc_ref0, acc_ref1, x_ref, root_ref, b_ref, o_ref):
        v = (acc_ref0[...] + acc_ref1[...]
             + jnp.dot(x_ref[...], root_ref[...],
                       preferred_element_type=jnp.float32)
             + b_ref[...])
        o_ref[...] = jnp.maximum(v, 0.0) if relu else v

    return pl.pallas_call(
        _body,
        grid=(_NBL,),
        in_specs=[
            pl.BlockSpec((_BL, D), lambda i: (i, 0)),
            pl.BlockSpec((_BL, D), lambda i: (_NBL + i, 0)),
            pl.BlockSpec((_BL, D), lambda i: (i, 0)),
            pl.BlockSpec((D, D), lambda i: (0, 0)),
            pl.BlockSpec((1, D), lambda i: (0, 0)),
        ],
        out_specs=pl.BlockSpec((_BL, D), lambda i: (i, 0)),
        out_shape=jax.ShapeDtypeStruct((N, D), jnp.float32),
    )


_combine_relu = _make_combine(True)
_combine_lin = _make_combine(False)


@jax.jit
def kernel(x, edge_index, edge_type, W0, root0, b0, W1, root1, b1):
    src = edge_index[0]
    dst = edge_index[1]
    npad = EPAD - E
    dst_a = jnp.concatenate(
        [dst, jnp.full((npad,), N, jnp.int32)]).reshape(EROWS, 128)
    dst_b = jnp.concatenate(
        [dst, jnp.zeros((npad,), jnp.int32)]).reshape(EROWS, 128)
    rel_p = jnp.concatenate(
        [edge_type, jnp.zeros((npad,), jnp.int32)]).reshape(EROWS, 128)
    src_p = jnp.concatenate(
        [src, jnp.zeros((npad,), jnp.int32)]).reshape(EROWS, 128)
    b0r = b0.reshape(1, D)
    b1r = b1.reshape(1, D)

    g2d, w2d = _phase_a(dst_a, rel_p, src_p)
    g64 = g2d.reshape(EROWS * 2, 64)
    w64 = w2d.reshape(EROWS * 2, 64)
    dst64 = dst_b.reshape(EROWS * 2, 64)

    h0 = _build_h(x, W0)
    acc0 = _phase_b(h0, g64, dst64, w64)
    x1 = _combine_relu(acc0, acc0, x, root0, b0r)

    h1 = _build_h(x1, W1)
    acc1 = _phase_b(h1, g64, dst64, w64)
    out = _combine_lin(acc1, acc1, x1, root1, b1r)
    return out
